# baseline (speedup 1.0000x reference)
import functools
import math

import jax
import jax.numpy as jnp
from jax import lax
from jax.experimental import pallas as pl
from jax.experimental.pallas import tpu as pltpu
from jax.experimental.pallas import tpu_sc as plsc

N = 50000
M = 16
NBR = 41
A = 64
LANES = 16

NW = 32
CHUNK_NODES = 32
CHUNK_EDGES = CHUNK_NODES * M
IDX_ROWS = CHUNK_EDGES // 128
CHUNKS_PER_WORKER = 49
NPW = CHUNKS_PER_WORKER * CHUNK_NODES
NPAD = NPW * NW
EPAD = NPAD * M
NCHUNKS = NPAD // CHUNK_NODES

BN = 512
BE = BN * M
GRID1 = NPAD // BN

PAIR_CHUNKS = 2 * CHUNKS_PER_WORKER
K0 = 59
MAXP = (max(K0, PAIR_CHUNKS - K0) + 1) // 2


def _bcast_lane(v, j):
    return lax.gather(
        v, jnp.full((LANES, 1), j, jnp.int32),
        lax.GatherDimensionNumbers(
            offset_dims=(), collapsed_slice_dims=(0,), start_index_map=(0,)),
        slice_sizes=(1,), mode=lax.GatherScatterMode.PROMISE_IN_BOUNDS)


def _softplus(x):
    return jnp.maximum(x, 0.0) + jnp.log1p(jnp.exp(-jnp.abs(x)))


def _prep_a_body(atom_t, nbr_t, wemb, bemb, wr1, br1, w2sel, b2bc, x0_o, w_o):
    x = lax.dot_general(atom_t[...], wemb[...], (((0,), (0,)), ((), ())),
                        preferred_element_type=jnp.float32)
    x0_o[...] = x + bemb[...]
    nb = nbr_t[...].reshape(NBR, BE)
    z = lax.dot_general(nb, wr1[...], (((0,), (0,)), ((), ())),
                        preferred_element_type=jnp.float32) + br1[...]
    s = jnp.log1p(jnp.exp(z))
    y = lax.dot_general(w2sel[...], s, (((0,), (1,)), ((), ())),
                        preferred_element_type=jnp.float32)
    scale = 1.0 / (M * math.sqrt(A))
    w_o[...] = ((y + b2bc[...]) * scale).reshape(M, BN)


def _prep_a_call(atom_t, nbr_t3, W_emb, b_emb2, wr1_0, br1_0r, w2sel0, b2bc0):
    return pl.pallas_call(
        _prep_a_body,
        grid=(GRID1,),
        in_specs=[
            pl.BlockSpec((atom_t.shape[0], BN), lambda i: (0, i)),
            pl.BlockSpec((NBR, M, BN), lambda i: (0, 0, i)),
            pl.BlockSpec(W_emb.shape, lambda i: (0, 0)),
            pl.BlockSpec(b_emb2.shape, lambda i: (0, 0)),
            pl.BlockSpec(wr1_0.shape, lambda i: (0, 0)),
            pl.BlockSpec(br1_0r.shape, lambda i: (0, 0)),
            pl.BlockSpec(w2sel0.shape, lambda i: (0, 0)),
            pl.BlockSpec((1, BE), lambda i: (0, 0)),
        ],
        out_specs=[
            pl.BlockSpec((BN, A), lambda i: (i, 0)),
            pl.BlockSpec((M, BN), lambda i: (0, i)),
        ],
        out_shape=[
            jax.ShapeDtypeStruct((NPAD, A), jnp.float32),
            jax.ShapeDtypeStruct((M, NPAD), jnp.float32),
        ],
    )(atom_t, nbr_t3, W_emb, b_emb2, wr1_0, br1_0r, w2sel0, b2bc0)


def _prep_b_body(nbr_t, wr1cat, br1cat, w2sel, b2bc, w_o):
    nb = nbr_t[...].reshape(NBR, BE)
    z = lax.dot_general(nb, wr1cat[...], (((0,), (0,)), ((), ())),
                        preferred_element_type=jnp.float32) + br1cat[...]
    s = jnp.log1p(jnp.exp(z))
    y2 = lax.dot_general(w2sel[...], s, (((0,), (1,)), ((), ())),
                         preferred_element_type=jnp.float32)
    scale = 1.0 / (M * math.sqrt(A))
    w_o[...] = ((y2 + b2bc[...]) * scale).reshape(2, M, BN)


def _prep_b_call(nbr_t3, wr1cat, br1cat, w2sel, b2bc):
    return pl.pallas_call(
        _prep_b_body,
        grid=(GRID1,),
        in_specs=[
            pl.BlockSpec((NBR, M, BN), lambda i: (0, 0, i)),
            pl.BlockSpec(wr1cat.shape, lambda i: (0, 0)),
            pl.BlockSpec(br1cat.shape, lambda i: (0, 0)),
            pl.BlockSpec(w2sel.shape, lambda i: (0, 0)),
            pl.BlockSpec((2, BE), lambda i: (0, 0)),
        ],
        out_specs=[
            pl.BlockSpec((2, M, BN), lambda i: (0, 0, i)),
        ],
        out_shape=[
            jax.ShapeDtypeStruct((2, M, NPAD), jnp.float32),
        ],
    )(nbr_t3, wr1cat, br1cat, w2sel, b2bc)


def _make_agg_kernel():
    mesh = plsc.VectorSubcoreMesh(core_axis_name="c", subcore_axis_name="s",
                                  num_cores=2, num_subcores=16)

    def body(table_h, idx_h, w_h, out_h,
             idx_v, w_v, rows_v, out_v,
             sg0, sg1, si0, si1, sw0, sw1, so0, so1):
        cid = lax.axis_index("c")
        sid = lax.axis_index("s")
        n = jnp.where(cid == 0, K0, PAIR_CHUNKS - K0)
        chunk0 = sid * PAIR_CHUNKS + jnp.where(cid == 0, 0, K0)

        sg = (sg0, sg1)
        si = (si0, si1)
        sw = (sw0, sw1)
        so = (so0, so1)

        def fire_gathers(b):
            for r in range(IDX_ROWS):
                pltpu.async_copy(
                    table_h.at[idx_v.at[b, r]],
                    rows_v.at[b, pl.ds(r * 128, 128)],
                    sg[b])

        def drain_gathers(b):
            for r in range(IDX_ROWS):
                pltpu.make_async_copy(
                    table_h.at[idx_v.at[b, r]],
                    rows_v.at[b, pl.ds(r * 128, 128)],
                    sg[b]).wait()

        for b in range(2):
            pltpu.sync_copy(idx_h.at[chunk0 + b], idx_v.at[b])
            fire_gathers(b)
            pltpu.async_copy(w_h.at[chunk0 + b], w_v.at[b], sw[b])

        def compute_chunk(b, cg):
            def node_body(nl, _):
                base = nl * M
                wrow = w_v[b, pl.ds(base, LANES)]
                acc = [jnp.zeros((LANES,), jnp.float32) for _ in range(A // LANES)]
                for j in range(M):
                    e = base + j
                    wj = _bcast_lane(wrow, j)
                    for q in range(A // LANES):
                        acc[q] = acc[q] + wj * rows_v[b, e, pl.ds(q * LANES, LANES)]
                for q in range(A // LANES):
                    out_v[b, nl, pl.ds(q * LANES, LANES)] = acc[q]
                return 0
            lax.fori_loop(0, CHUNK_NODES, node_body, 0)
            pltpu.async_copy(
                out_v.at[b],
                out_h.at[pl.ds(cg * CHUNK_NODES, CHUNK_NODES)],
                so[b])

        def wait_out(b, cg):
            pltpu.make_async_copy(
                out_v.at[b],
                out_h.at[pl.ds(cg * CHUNK_NODES, CHUNK_NODES)],
                so[b]).wait()

        def outer(it, carry):
            for b in range(2):
                c = it * 2 + b
                cg = chunk0 + c

                @pl.when(c < n)
                def _():
                    drain_gathers(b)

                    @pl.when(c + 2 < n)
                    def _():
                        pltpu.async_copy(idx_h.at[cg + 2], idx_v.at[b], si[b])

                    pltpu.make_async_copy(w_h.at[cg], w_v.at[b], sw[b]).wait()

                    @pl.when(c >= 2)
                    def _():
                        wait_out(b, cg - 2)

                    compute_chunk(b, cg)

                    @pl.when(c + 2 < n)
                    def _():
                        pltpu.make_async_copy(
                            idx_h.at[cg + 2], idx_v.at[b], si[b]).wait()
                        fire_gathers(b)
                        pltpu.async_copy(w_h.at[cg + 2], w_v.at[b], sw[b])

            return carry

        lax.fori_loop(0, MAXP, outer, 0)
        wait_out(0, chunk0)
        wait_out(1, chunk0)

    kern = pl.kernel(
        body,
        out_type=jax.ShapeDtypeStruct((NPAD, A), jnp.float32),
        mesh=mesh,
        scratch_types=[
            pltpu.VMEM((2, IDX_ROWS, 128), jnp.int32),
            pltpu.VMEM((2, CHUNK_EDGES), jnp.float32),
            pltpu.VMEM((2, CHUNK_EDGES, A), jnp.float32),
            pltpu.VMEM((2, CHUNK_NODES, A), jnp.float32),
        ] + [pltpu.SemaphoreType.DMA] * 8,
        compiler_params=pltpu.CompilerParams(use_tc_tiling_on_sc=False),
    )
    return kern


def _make_pool_kernel(n_cry, ap):
    mesh = plsc.VectorSubcoreMesh(core_axis_name="c", subcore_axis_name="s",
                                  num_cores=2, num_subcores=16)
    cpw = n_cry // NW
    ipw = cpw * ap
    rows128 = ipw // 128

    def body(table_h, idx_h, out_h, idx_v, rows_v, out_v, sg):
        cid = lax.axis_index("c")
        sid = lax.axis_index("s")
        wid = sid * 2 + cid
        pltpu.sync_copy(idx_h.at[wid], idx_v)
        for r in range(rows128):
            pltpu.async_copy(
                table_h.at[idx_v.at[r]],
                rows_v.at[pl.ds(r * 128, 128)], sg)
        for r in range(rows128):
            pltpu.make_async_copy(
                table_h.at[idx_v.at[r]],
                rows_v.at[pl.ds(r * 128, 128)], sg).wait()
        inv = 1.0 / ap

        def cry_body(k, _):
            base = k * ap
            acc = [jnp.zeros((LANES,), jnp.float32) for _ in range(A // LANES)]
            for j in range(ap):
                for q in range(A // LANES):
                    acc[q] = acc[q] + rows_v[base + j, pl.ds(q * LANES, LANES)]
            for q in range(A // LANES):
                out_v[k, pl.ds(q * LANES, LANES)] = acc[q] * inv
            return 0
        lax.fori_loop(0, cpw, cry_body, 0)
        pltpu.sync_copy(out_v, out_h.at[pl.ds(wid * cpw, cpw)])

    kern = pl.kernel(
        body,
        out_type=jax.ShapeDtypeStruct((n_cry, A), jnp.float32),
        mesh=mesh,
        scratch_types=[
            pltpu.VMEM((rows128, 128), jnp.int32),
            pltpu.VMEM((ipw, A), jnp.float32),
            pltpu.VMEM((cpw, A), jnp.float32),
            pltpu.SemaphoreType.DMA,
        ],
        compiler_params=pltpu.CompilerParams(use_tc_tiling_on_sc=False),
    )
    return kern


def _head_body(pool, wt0, wt1, wt2, wfc, bfc, wout, bout, out_o, h_o):
    wc = jnp.dot(jnp.dot(wt0[...], wt1[...], preferred_element_type=jnp.float32),
                 wt2[...], preferred_element_type=jnp.float32)
    crys = jnp.dot(pool[...], wc, preferred_element_type=jnp.float32)
    pre = jnp.dot(crys, wfc[...], preferred_element_type=jnp.float32) + bfc[...]
    h = _softplus(pre)
    h_o[...] = h
    out_o[...] = jnp.dot(h, wout[...], preferred_element_type=jnp.float32) + bout[...]


def _head_call(pool, W_tp_0, W_tp_1, W_tp_2, W_fc, b_fc2, W_out, b_out2):
    n_cry = pool.shape[0]
    H = W_fc.shape[1]
    return pl.pallas_call(
        _head_body,
        out_shape=[
            jax.ShapeDtypeStruct((n_cry, 1), jnp.float32),
            jax.ShapeDtypeStruct((n_cry, H), jnp.float32),
        ],
    )(pool, W_tp_0, W_tp_1, W_tp_2, W_fc, b_fc2, W_out, b_out2)


def kernel(atom_fea, nbr_fea, nbr_idx, crystal_atom_idx, W_emb, b_emb,
           W_r1_0, b_r1_0, W_r2_0, b_r2_0, W_tp_0,
           W_r1_1, b_r1_1, W_r2_1, b_r2_1, W_tp_1,
           W_r1_2, b_r1_2, W_r2_2, b_r2_2, W_tp_2,
           W_fc, b_fc, W_out, b_out):
    n, m = nbr_idx.shape
    n_cry, ap = crystal_atom_idx.shape

    atom_t = atom_fea.T
    nbr_t3 = jnp.transpose(nbr_fea, (2, 1, 0))
    br1_0r = b_r1_0.reshape(1, NBR)
    w2sel0 = W_r2_0[:, 0].reshape(NBR, 1)
    wr1cat = jnp.concatenate([W_r1_1, W_r1_2], axis=1)
    br1cat = jnp.concatenate([b_r1_1, b_r1_2]).reshape(1, 2 * NBR)
    w2sel = jnp.zeros((2 * NBR, 2), jnp.float32)
    w2sel = w2sel.at[0 * NBR:1 * NBR, 0].set(W_r2_1[:, 0])
    w2sel = w2sel.at[1 * NBR:2 * NBR, 1].set(W_r2_2[:, 0])
    b2bc0 = jnp.broadcast_to(b_r2_0[0].reshape(1, 1), (1, BE))
    b2bc12 = jnp.broadcast_to(
        jnp.stack([b_r2_1[0], b_r2_2[0]]).reshape(2, 1), (2, BE))
    b_emb2 = b_emb.reshape(1, A)

    x0, w0_2d = _prep_a_call(atom_t, nbr_t3, W_emb, b_emb2,
                             W_r1_0, br1_0r, w2sel0, b2bc0)
    (w12_3d,) = _prep_b_call(nbr_t3, wr1cat, br1cat, w2sel, b2bc12)

    w0_node = w0_2d.T.reshape(EPAD)
    w12_node = w12_3d.transpose(0, 2, 1).reshape(2, EPAD)
    idx_flat = nbr_idx.reshape(-1)
    idx_pad = jnp.pad(idx_flat, (0, EPAD - n * m)).reshape(NCHUNKS, IDX_ROWS, 128)
    w_chunks = [w0_node.reshape(NCHUNKS, CHUNK_EDGES),
                w12_node[0].reshape(NCHUNKS, CHUNK_EDGES),
                w12_node[1].reshape(NCHUNKS, CHUNK_EDGES)]

    agg = _make_agg_kernel()
    t = x0
    for l in range(3):
        t = agg(t, idx_pad, w_chunks[l])

    cry_idx = crystal_atom_idx.reshape(NW, (n_cry * ap) // (NW * 128), 128)
    pool = _make_pool_kernel(n_cry, ap)(t, cry_idx)

    out, h = _head_call(pool, W_tp_0, W_tp_1, W_tp_2, W_fc,
                        b_fc.reshape(1, -1), W_out, b_out.reshape(1, 1))
    return (out, h)


# speedup vs baseline: 1.1253x; 1.1253x over previous
import functools
import math

import jax
import jax.numpy as jnp
from jax import lax
from jax.experimental import pallas as pl
from jax.experimental.pallas import tpu as pltpu
from jax.experimental.pallas import tpu_sc as plsc

N = 50000
M = 16
NBR = 41
A = 64
LANES = 16

NW = 32
CHUNK_NODES = 32
CHUNK_EDGES = CHUNK_NODES * M
IDX_ROWS = CHUNK_EDGES // 128
CHUNKS_PER_WORKER = 49
NPW = CHUNKS_PER_WORKER * CHUNK_NODES
NPAD = NPW * NW
EPAD = NPAD * M
NCHUNKS = NPAD // CHUNK_NODES

BN = 512
BE = BN * M
GRID1 = NPAD // BN

PAIR_CHUNKS = 2 * CHUNKS_PER_WORKER
K0 = 49
MAXP = (max(K0, PAIR_CHUNKS - K0) + 1) // 2


A32 = 32


def _unpack_bf16(u):
    lo = plsc.bitcast(u << 16, jnp.float32)
    hi = plsc.bitcast(u & jnp.int32(-65536), jnp.float32)
    return lo, hi


def _pack_bf16(lo, hi):
    ve = plsc.bitcast(lo, jnp.int32)
    re = ve + (jnp.int32(0x7FFF) + ((ve >> 16) & 1))
    vo = plsc.bitcast(hi, jnp.int32)
    ro = vo + (jnp.int32(0x7FFF) + ((vo >> 16) & 1))
    return lax.shift_right_logical(re, 16) | (ro & jnp.int32(-65536))


def _bcast_lane(v, j):
    return lax.gather(
        v, jnp.full((LANES, 1), j, jnp.int32),
        lax.GatherDimensionNumbers(
            offset_dims=(), collapsed_slice_dims=(0,), start_index_map=(0,)),
        slice_sizes=(1,), mode=lax.GatherScatterMode.PROMISE_IN_BOUNDS)


def _softplus(x):
    return jnp.maximum(x, 0.0) + jnp.log1p(jnp.exp(-jnp.abs(x)))


def _prep_body(atom_t, nbr_t, wemb, bemb, wr1cat, br1cat, w2sel, b2bc, x0_o, w_o):
    x = lax.dot_general(atom_t[...], wemb[...], (((0,), (0,)), ((), ())),
                        preferred_element_type=jnp.float32)
    x0_o[...] = (x + bemb[...]).astype(jnp.bfloat16)
    nb = nbr_t[...].reshape(NBR, BE)
    z = lax.dot_general(nb, wr1cat[...], (((0,), (0,)), ((), ())),
                        preferred_element_type=jnp.float32) + br1cat[...]
    s = jnp.log1p(jnp.exp(z))
    y3 = lax.dot_general(w2sel[...], s, (((0,), (1,)), ((), ())),
                         preferred_element_type=jnp.float32)
    scale = 1.0 / (M * math.sqrt(A))
    w_o[...] = ((y3 + b2bc[...]) * scale).reshape(3, M, BN)


def _prep_call(atom_t, nbr_t3, W_emb, b_emb2, wr1cat, br1cat, w2sel, b2bc):
    return pl.pallas_call(
        _prep_body,
        grid=(GRID1,),
        in_specs=[
            pl.BlockSpec((atom_t.shape[0], BN), lambda i: (0, i)),
            pl.BlockSpec((NBR, M, BN), lambda i: (0, 0, i)),
            pl.BlockSpec(W_emb.shape, lambda i: (0, 0)),
            pl.BlockSpec(b_emb2.shape, lambda i: (0, 0)),
            pl.BlockSpec(wr1cat.shape, lambda i: (0, 0)),
            pl.BlockSpec(br1cat.shape, lambda i: (0, 0)),
            pl.BlockSpec(w2sel.shape, lambda i: (0, 0)),
            pl.BlockSpec((3, BE), lambda i: (0, 0)),
        ],
        out_specs=[
            pl.BlockSpec((BN, A), lambda i: (i, 0)),
            pl.BlockSpec((3, M, BN), lambda i: (0, 0, i)),
        ],
        out_shape=[
            jax.ShapeDtypeStruct((NPAD, A), jnp.bfloat16),
            jax.ShapeDtypeStruct((3, M, NPAD), jnp.float32),
        ],
    )(atom_t, nbr_t3, W_emb, b_emb2, wr1cat, br1cat, w2sel, b2bc)


def _make_agg_kernel():
    mesh = plsc.VectorSubcoreMesh(core_axis_name="c", subcore_axis_name="s",
                                  num_cores=2, num_subcores=16)

    def body(table_h, idx_h, w_h, out_h,
             idx_v, w_v, rows_v, out_v,
             sg0, sg1, si0, si1, sw0, sw1, so0, so1):
        cid = lax.axis_index("c")
        sid = lax.axis_index("s")
        n = jnp.where(cid == 0, K0, PAIR_CHUNKS - K0)
        chunk0 = sid * PAIR_CHUNKS + jnp.where(cid == 0, 0, K0)

        sg = (sg0, sg1)
        si = (si0, si1)
        sw = (sw0, sw1)
        so = (so0, so1)

        def fire_gathers(b):
            for r in range(IDX_ROWS):
                pltpu.async_copy(
                    table_h.at[idx_v.at[b, r]],
                    rows_v.at[b, pl.ds(r * 128, 128)],
                    sg[b])

        def drain_gathers(b):
            for r in range(IDX_ROWS):
                pltpu.make_async_copy(
                    table_h.at[idx_v.at[b, r]],
                    rows_v.at[b, pl.ds(r * 128, 128)],
                    sg[b]).wait()

        for b in range(2):
            pltpu.sync_copy(idx_h.at[chunk0 + b], idx_v.at[b])
            fire_gathers(b)
            pltpu.async_copy(w_h.at[chunk0 + b], w_v.at[b], sw[b])

        def compute_chunk(b, cg):
            def node_body(nl, _):
                base = nl * M
                wrow = w_v[b, pl.ds(base, LANES)]
                acc = [jnp.zeros((LANES,), jnp.float32) for _ in range(2 * (A32 // LANES))]
                for j in range(M):
                    e = base + j
                    wj = _bcast_lane(wrow, j)
                    for h in range(A32 // LANES):
                        lo, hi = _unpack_bf16(rows_v[b, e, pl.ds(h * LANES, LANES)])
                        acc[2 * h] = acc[2 * h] + wj * lo
                        acc[2 * h + 1] = acc[2 * h + 1] + wj * hi
                for h in range(A32 // LANES):
                    out_v[b, nl, pl.ds(h * LANES, LANES)] = _pack_bf16(
                        acc[2 * h], acc[2 * h + 1])
                return 0
            lax.fori_loop(0, CHUNK_NODES, node_body, 0)
            pltpu.async_copy(
                out_v.at[b],
                out_h.at[pl.ds(cg * CHUNK_NODES, CHUNK_NODES)],
                so[b])

        def wait_out(b, cg):
            pltpu.make_async_copy(
                out_v.at[b],
                out_h.at[pl.ds(cg * CHUNK_NODES, CHUNK_NODES)],
                so[b]).wait()

        def outer(it, carry):
            for b in range(2):
                c = it * 2 + b
                cg = chunk0 + c

                @pl.when(c < n)
                def _():
                    drain_gathers(b)

                    @pl.when(c + 2 < n)
                    def _():
                        pltpu.async_copy(idx_h.at[cg + 2], idx_v.at[b], si[b])

                    pltpu.make_async_copy(w_h.at[cg], w_v.at[b], sw[b]).wait()

                    @pl.when(c >= 2)
                    def _():
                        wait_out(b, cg - 2)

                    compute_chunk(b, cg)

                    @pl.when(c + 2 < n)
                    def _():
                        pltpu.make_async_copy(
                            idx_h.at[cg + 2], idx_v.at[b], si[b]).wait()
                        fire_gathers(b)
                        pltpu.async_copy(w_h.at[cg + 2], w_v.at[b], sw[b])

            return carry

        lax.fori_loop(0, MAXP, outer, 0)
        wait_out(0, chunk0)
        wait_out(1, chunk0)

    kern = pl.kernel(
        body,
        out_type=jax.ShapeDtypeStruct((NPAD, A32), jnp.int32),
        mesh=mesh,
        scratch_types=[
            pltpu.VMEM((2, IDX_ROWS, 128), jnp.int32),
            pltpu.VMEM((2, CHUNK_EDGES), jnp.float32),
            pltpu.VMEM((2, CHUNK_EDGES, A32), jnp.int32),
            pltpu.VMEM((2, CHUNK_NODES, A32), jnp.int32),
        ] + [pltpu.SemaphoreType.DMA] * 8,
        compiler_params=pltpu.CompilerParams(use_tc_tiling_on_sc=False,
                                             needs_layout_passes=False),
    )
    return kern


def _make_pool_kernel(n_cry, ap):
    mesh = plsc.VectorSubcoreMesh(core_axis_name="c", subcore_axis_name="s",
                                  num_cores=2, num_subcores=16)
    cpw = n_cry // NW
    ipw = cpw * ap
    rows128 = ipw // 128

    def body(table_h, idx_h, out_h, idx_v, rows_v, out_v, sg):
        cid = lax.axis_index("c")
        sid = lax.axis_index("s")
        wid = sid * 2 + cid
        pltpu.sync_copy(idx_h.at[wid], idx_v)
        for r in range(rows128):
            pltpu.async_copy(
                table_h.at[idx_v.at[r]],
                rows_v.at[pl.ds(r * 128, 128)], sg)
        for r in range(rows128):
            pltpu.make_async_copy(
                table_h.at[idx_v.at[r]],
                rows_v.at[pl.ds(r * 128, 128)], sg).wait()
        inv = 1.0 / ap

        def cry_body(k, _):
            base = k * ap
            acc = [jnp.zeros((LANES,), jnp.float32) for _ in range(2 * (A32 // LANES))]
            for j in range(ap):
                for h in range(A32 // LANES):
                    lo, hi = _unpack_bf16(rows_v[base + j, pl.ds(h * LANES, LANES)])
                    acc[2 * h] = acc[2 * h] + lo
                    acc[2 * h + 1] = acc[2 * h + 1] + hi
            for h in range(A32 // LANES):
                out_v[k, pl.ds(h * LANES, LANES)] = acc[2 * h] * inv
                out_v[k, pl.ds(A // 2 + h * LANES, LANES)] = acc[2 * h + 1] * inv
            return 0
        lax.fori_loop(0, cpw, cry_body, 0)
        pltpu.sync_copy(out_v, out_h.at[pl.ds(wid * cpw, cpw)])

    kern = pl.kernel(
        body,
        out_type=jax.ShapeDtypeStruct((n_cry, A), jnp.float32),
        mesh=mesh,
        scratch_types=[
            pltpu.VMEM((rows128, 128), jnp.int32),
            pltpu.VMEM((ipw, A32), jnp.int32),
            pltpu.VMEM((cpw, A), jnp.float32),
            pltpu.SemaphoreType.DMA,
        ],
        compiler_params=pltpu.CompilerParams(use_tc_tiling_on_sc=False,
                                             needs_layout_passes=False),
    )
    return kern


def _head_body(pool, wt0, wt1, wt2, wfc, bfc, wout, bout, out_o, h_o):
    wc = jnp.dot(jnp.dot(wt0[...], wt1[...], preferred_element_type=jnp.float32),
                 wt2[...], preferred_element_type=jnp.float32)
    crys = jnp.dot(pool[...].astype(jnp.float32), wc,
                   preferred_element_type=jnp.float32)
    pre = jnp.dot(crys, wfc[...], preferred_element_type=jnp.float32) + bfc[...]
    h = _softplus(pre)
    h_o[...] = h
    out_o[...] = jnp.dot(h, wout[...], preferred_element_type=jnp.float32) + bout[...]


def _head_call(pool, W_tp_0, W_tp_1, W_tp_2, W_fc, b_fc2, W_out, b_out2):
    n_cry = pool.shape[0]
    H = W_fc.shape[1]
    return pl.pallas_call(
        _head_body,
        out_shape=[
            jax.ShapeDtypeStruct((n_cry, 1), jnp.float32),
            jax.ShapeDtypeStruct((n_cry, H), jnp.float32),
        ],
    )(pool, W_tp_0, W_tp_1, W_tp_2, W_fc, b_fc2, W_out, b_out2)


def kernel(atom_fea, nbr_fea, nbr_idx, crystal_atom_idx, W_emb, b_emb,
           W_r1_0, b_r1_0, W_r2_0, b_r2_0, W_tp_0,
           W_r1_1, b_r1_1, W_r2_1, b_r2_1, W_tp_1,
           W_r1_2, b_r1_2, W_r2_2, b_r2_2, W_tp_2,
           W_fc, b_fc, W_out, b_out):
    n, m = nbr_idx.shape
    n_cry, ap = crystal_atom_idx.shape

    atom_t = atom_fea.T
    nbr_t3 = jnp.transpose(nbr_fea, (2, 1, 0))
    wr1cat = jnp.concatenate([W_r1_0, W_r1_1, W_r1_2], axis=1)
    br1cat = jnp.concatenate([b_r1_0, b_r1_1, b_r1_2]).reshape(1, 3 * NBR)
    w2sel = jnp.zeros((3 * NBR, 3), jnp.float32)
    w2sel = w2sel.at[0 * NBR:1 * NBR, 0].set(W_r2_0[:, 0])
    w2sel = w2sel.at[1 * NBR:2 * NBR, 1].set(W_r2_1[:, 0])
    w2sel = w2sel.at[2 * NBR:3 * NBR, 2].set(W_r2_2[:, 0])
    b2bc = jnp.broadcast_to(
        jnp.stack([b_r2_0[0], b_r2_1[0], b_r2_2[0]]).reshape(3, 1), (3, BE))
    b_emb2 = b_emb.reshape(1, A)

    x0, w3d = _prep_call(atom_t, nbr_t3, W_emb, b_emb2, wr1cat, br1cat, w2sel, b2bc)

    w_node = w3d.transpose(0, 2, 1).reshape(3, EPAD)
    idx_flat = nbr_idx.reshape(-1)
    idx_pad = jnp.pad(idx_flat, (0, EPAD - n * m)).reshape(NCHUNKS, IDX_ROWS, 128)
    w_chunks = [w_node[l].reshape(NCHUNKS, CHUNK_EDGES) for l in range(3)]

    x0_i = lax.bitcast_convert_type(x0.reshape(NPAD, A32, 2), jnp.int32)

    agg = _make_agg_kernel()
    t = x0_i
    for l in range(3):
        t = agg(t, idx_pad, w_chunks[l])

    cry_idx = crystal_atom_idx.reshape(NW, (n_cry * ap) // (NW * 128), 128)
    pool_perm = _make_pool_kernel(n_cry, ap)(t, cry_idx)
    pool = pool_perm.reshape(n_cry, 2, 2, LANES).transpose(0, 2, 3, 1).reshape(n_cry, A)

    out, h = _head_call(pool, W_tp_0, W_tp_1, W_tp_2, W_fc,
                        b_fc.reshape(1, -1), W_out, b_out.reshape(1, 1))
    return (out, h)


# speedup vs baseline: 1.1925x; 1.0597x over previous
import functools
import math

import jax
import jax.numpy as jnp
from jax import lax
from jax.experimental import pallas as pl
from jax.experimental.pallas import tpu as pltpu
from jax.experimental.pallas import tpu_sc as plsc

N = 50000
M = 16
NBR = 41
A = 64
LANES = 16

NW = 32
CHUNK_NODES = 32
CHUNK_EDGES = CHUNK_NODES * M
IDX_ROWS = CHUNK_EDGES // 128
CHUNKS_PER_WORKER = 49
NPW = CHUNKS_PER_WORKER * CHUNK_NODES
NPAD = NPW * NW
EPAD = NPAD * M
NCHUNKS = NPAD // CHUNK_NODES

BN = 512
BE = BN * M
GRID1 = NPAD // BN

PAIR_CHUNKS = 2 * CHUNKS_PER_WORKER
K0 = 49
MAXP = (max(K0, PAIR_CHUNKS - K0) + 1) // 2


A32 = 32


def _unpack_bf16(u):
    lo = plsc.bitcast(u << 16, jnp.float32)
    hi = plsc.bitcast(u & jnp.int32(-65536), jnp.float32)
    return lo, hi


def _pack_bf16(lo, hi):
    ve = plsc.bitcast(lo, jnp.int32)
    re = ve + (jnp.int32(0x7FFF) + ((ve >> 16) & 1))
    vo = plsc.bitcast(hi, jnp.int32)
    ro = vo + (jnp.int32(0x7FFF) + ((vo >> 16) & 1))
    return lax.shift_right_logical(re, 16) | (ro & jnp.int32(-65536))


def _bcast_lane(v, j):
    return lax.gather(
        v, jnp.full((LANES, 1), j, jnp.int32),
        lax.GatherDimensionNumbers(
            offset_dims=(), collapsed_slice_dims=(0,), start_index_map=(0,)),
        slice_sizes=(1,), mode=lax.GatherScatterMode.PROMISE_IN_BOUNDS)


def _softplus(x):
    return jnp.maximum(x, 0.0) + jnp.log1p(jnp.exp(-jnp.abs(x)))


def _prep_body(atom_t, nbr_t, wemb, bemb, wr1cat, br1cat, w2sel, b2bc, x0_o, w_o):
    x = lax.dot_general(atom_t[...], wemb[...], (((0,), (0,)), ((), ())),
                        preferred_element_type=jnp.float32)
    x0_o[...] = x + bemb[...]
    nb = nbr_t[...].reshape(NBR, BE)
    z = lax.dot_general(nb, wr1cat[...], (((0,), (0,)), ((), ())),
                        preferred_element_type=jnp.float32) + br1cat[...]
    s = jnp.log1p(jnp.exp(z))
    y3 = lax.dot_general(w2sel[...], s, (((0,), (1,)), ((), ())),
                         preferred_element_type=jnp.float32)
    scale = 1.0 / (M * math.sqrt(A))
    w_o[...] = ((y3 + b2bc[...]) * scale).reshape(3, M, BN)


def _prep_call(atom_t, nbr_t3, W_emb, b_emb2, wr1cat, br1cat, w2sel, b2bc):
    return pl.pallas_call(
        _prep_body,
        grid=(GRID1,),
        in_specs=[
            pl.BlockSpec((atom_t.shape[0], BN), lambda i: (0, i)),
            pl.BlockSpec((NBR, M, BN), lambda i: (0, 0, i)),
            pl.BlockSpec(W_emb.shape, lambda i: (0, 0)),
            pl.BlockSpec(b_emb2.shape, lambda i: (0, 0)),
            pl.BlockSpec(wr1cat.shape, lambda i: (0, 0)),
            pl.BlockSpec(br1cat.shape, lambda i: (0, 0)),
            pl.BlockSpec(w2sel.shape, lambda i: (0, 0)),
            pl.BlockSpec((3, BE), lambda i: (0, 0)),
        ],
        out_specs=[
            pl.BlockSpec((BN, A), lambda i: (i, 0)),
            pl.BlockSpec((3, M, BN), lambda i: (0, 0, i)),
        ],
        out_shape=[
            jax.ShapeDtypeStruct((NPAD, A), jnp.float32),
            jax.ShapeDtypeStruct((3, M, NPAD), jnp.float32),
        ],
    )(atom_t, nbr_t3, W_emb, b_emb2, wr1cat, br1cat, w2sel, b2bc)


def _make_agg_kernel(f32_in):
    mesh = plsc.VectorSubcoreMesh(core_axis_name="c", subcore_axis_name="s",
                                  num_cores=2, num_subcores=16)
    row_w = A if f32_in else A32

    def body(table_h, idx_h, w_h, out_h,
             idx_v, w_v, rows_v, out_v,
             sg0, sg1, si0, si1, sw0, sw1, so0, so1):
        cid = lax.axis_index("c")
        sid = lax.axis_index("s")
        n = jnp.where(cid == 0, K0, PAIR_CHUNKS - K0)
        chunk0 = sid * PAIR_CHUNKS + jnp.where(cid == 0, 0, K0)

        sg = (sg0, sg1)
        si = (si0, si1)
        sw = (sw0, sw1)
        so = (so0, so1)

        def fire_gathers(b):
            for r in range(IDX_ROWS):
                pltpu.async_copy(
                    table_h.at[idx_v.at[b, r]],
                    rows_v.at[b, pl.ds(r * 128, 128)],
                    sg[b])

        def drain_gathers(b):
            for r in range(IDX_ROWS):
                pltpu.make_async_copy(
                    table_h.at[idx_v.at[b, r]],
                    rows_v.at[b, pl.ds(r * 128, 128)],
                    sg[b]).wait()

        for b in range(2):
            pltpu.sync_copy(idx_h.at[chunk0 + b], idx_v.at[b])
            fire_gathers(b)
            pltpu.async_copy(w_h.at[chunk0 + b], w_v.at[b], sw[b])

        def compute_chunk(b, cg):
            def node_body(nl, _):
                base = nl * M
                wrow = w_v[b, pl.ds(base, LANES)]
                acc = [jnp.zeros((LANES,), jnp.float32) for _ in range(A // LANES)]
                for j in range(M):
                    e = base + j
                    wj = _bcast_lane(wrow, j)
                    if f32_in:
                        for q in range(A // LANES):
                            acc[q] = acc[q] + wj * rows_v[b, e, pl.ds(q * LANES, LANES)]
                    else:
                        for h in range(A32 // LANES):
                            lo, hi = _unpack_bf16(rows_v[b, e, pl.ds(h * LANES, LANES)])
                            acc[2 * h] = acc[2 * h] + wj * lo
                            acc[2 * h + 1] = acc[2 * h + 1] + wj * hi
                for h in range(A32 // LANES):
                    out_v[b, nl, pl.ds(h * LANES, LANES)] = _pack_bf16(
                        acc[2 * h], acc[2 * h + 1])
                return 0
            lax.fori_loop(0, CHUNK_NODES, node_body, 0)
            pltpu.async_copy(
                out_v.at[b],
                out_h.at[pl.ds(cg * CHUNK_NODES, CHUNK_NODES)],
                so[b])

        def wait_out(b, cg):
            pltpu.make_async_copy(
                out_v.at[b],
                out_h.at[pl.ds(cg * CHUNK_NODES, CHUNK_NODES)],
                so[b]).wait()

        def outer(it, carry):
            for b in range(2):
                c = it * 2 + b
                cg = chunk0 + c

                @pl.when(c < n)
                def _():
                    drain_gathers(b)

                    @pl.when(c + 2 < n)
                    def _():
                        pltpu.async_copy(idx_h.at[cg + 2], idx_v.at[b], si[b])

                    pltpu.make_async_copy(w_h.at[cg], w_v.at[b], sw[b]).wait()

                    @pl.when(c >= 2)
                    def _():
                        wait_out(b, cg - 2)

                    compute_chunk(b, cg)

                    @pl.when(c + 2 < n)
                    def _():
                        pltpu.make_async_copy(
                            idx_h.at[cg + 2], idx_v.at[b], si[b]).wait()
                        fire_gathers(b)
                        pltpu.async_copy(w_h.at[cg + 2], w_v.at[b], sw[b])

            return carry

        lax.fori_loop(0, MAXP, outer, 0)
        wait_out(0, chunk0)
        wait_out(1, chunk0)

    kern = pl.kernel(
        body,
        out_type=jax.ShapeDtypeStruct((NPAD, A32), jnp.int32),
        mesh=mesh,
        scratch_types=[
            pltpu.VMEM((2, IDX_ROWS, 128), jnp.int32),
            pltpu.VMEM((2, CHUNK_EDGES), jnp.float32),
            pltpu.VMEM((2, CHUNK_EDGES, row_w),
                       jnp.float32 if f32_in else jnp.int32),
            pltpu.VMEM((2, CHUNK_NODES, A32), jnp.int32),
        ] + [pltpu.SemaphoreType.DMA] * 8,
        compiler_params=pltpu.CompilerParams(use_tc_tiling_on_sc=False,
                                             needs_layout_passes=False),
    )
    return kern


def _make_pool_kernel(n_cry, ap):
    mesh = plsc.VectorSubcoreMesh(core_axis_name="c", subcore_axis_name="s",
                                  num_cores=2, num_subcores=16)
    cpw = n_cry // NW
    ipw = cpw * ap
    rows128 = ipw // 128

    def body(table_h, idx_h, out_h, idx_v, rows_v, out_v, sg):
        cid = lax.axis_index("c")
        sid = lax.axis_index("s")
        wid = sid * 2 + cid
        pltpu.sync_copy(idx_h.at[wid], idx_v)
        for r in range(rows128):
            pltpu.async_copy(
                table_h.at[idx_v.at[r]],
                rows_v.at[pl.ds(r * 128, 128)], sg)
        for r in range(rows128):
            pltpu.make_async_copy(
                table_h.at[idx_v.at[r]],
                rows_v.at[pl.ds(r * 128, 128)], sg).wait()
        inv = 1.0 / ap

        def cry_body(k, _):
            base = k * ap
            acc = [jnp.zeros((LANES,), jnp.float32) for _ in range(2 * (A32 // LANES))]
            for j in range(ap):
                for h in range(A32 // LANES):
                    lo, hi = _unpack_bf16(rows_v[base + j, pl.ds(h * LANES, LANES)])
                    acc[2 * h] = acc[2 * h] + lo
                    acc[2 * h + 1] = acc[2 * h + 1] + hi
            for i in range(A // LANES):
                out_v[k, pl.ds(i * LANES, LANES)] = acc[i] * inv
            return 0
        lax.fori_loop(0, cpw, cry_body, 0)
        pltpu.sync_copy(out_v, out_h.at[pl.ds(wid * cpw, cpw)])

    kern = pl.kernel(
        body,
        out_type=jax.ShapeDtypeStruct((n_cry, A), jnp.float32),
        mesh=mesh,
        scratch_types=[
            pltpu.VMEM((rows128, 128), jnp.int32),
            pltpu.VMEM((ipw, A32), jnp.int32),
            pltpu.VMEM((cpw, A), jnp.float32),
            pltpu.SemaphoreType.DMA,
        ],
        compiler_params=pltpu.CompilerParams(use_tc_tiling_on_sc=False,
                                             needs_layout_passes=False),
    )
    return kern


def _head_body(pool, wt0, wt1, wt2, wfc, bfc, wout, bout, out_o, h_o):
    wc = jnp.dot(jnp.dot(wt0[...], wt1[...], preferred_element_type=jnp.float32),
                 wt2[...], preferred_element_type=jnp.float32)
    crys = jnp.dot(pool[...].astype(jnp.float32), wc,
                   preferred_element_type=jnp.float32)
    pre = jnp.dot(crys, wfc[...], preferred_element_type=jnp.float32) + bfc[...]
    h = _softplus(pre)
    h_o[...] = h
    out_o[...] = jnp.dot(h, wout[...], preferred_element_type=jnp.float32) + bout[...]


def _head_call(pool, W_tp_0, W_tp_1, W_tp_2, W_fc, b_fc2, W_out, b_out2):
    n_cry = pool.shape[0]
    H = W_fc.shape[1]
    return pl.pallas_call(
        _head_body,
        out_shape=[
            jax.ShapeDtypeStruct((n_cry, 1), jnp.float32),
            jax.ShapeDtypeStruct((n_cry, H), jnp.float32),
        ],
    )(pool, W_tp_0, W_tp_1, W_tp_2, W_fc, b_fc2, W_out, b_out2)


def kernel(atom_fea, nbr_fea, nbr_idx, crystal_atom_idx, W_emb, b_emb,
           W_r1_0, b_r1_0, W_r2_0, b_r2_0, W_tp_0,
           W_r1_1, b_r1_1, W_r2_1, b_r2_1, W_tp_1,
           W_r1_2, b_r1_2, W_r2_2, b_r2_2, W_tp_2,
           W_fc, b_fc, W_out, b_out):
    n, m = nbr_idx.shape
    n_cry, ap = crystal_atom_idx.shape

    atom_t = atom_fea.T
    nbr_t3 = jnp.transpose(nbr_fea, (2, 1, 0))
    wr1cat = jnp.concatenate([W_r1_0, W_r1_1, W_r1_2], axis=1)
    br1cat = jnp.concatenate([b_r1_0, b_r1_1, b_r1_2]).reshape(1, 3 * NBR)
    w2sel = jnp.zeros((3 * NBR, 3), jnp.float32)
    w2sel = w2sel.at[0 * NBR:1 * NBR, 0].set(W_r2_0[:, 0])
    w2sel = w2sel.at[1 * NBR:2 * NBR, 1].set(W_r2_1[:, 0])
    w2sel = w2sel.at[2 * NBR:3 * NBR, 2].set(W_r2_2[:, 0])
    b2bc = jnp.broadcast_to(
        jnp.stack([b_r2_0[0], b_r2_1[0], b_r2_2[0]]).reshape(3, 1), (3, BE))
    b_emb2 = b_emb.reshape(1, A)

    x0, w3d = _prep_call(atom_t, nbr_t3, W_emb, b_emb2, wr1cat, br1cat, w2sel, b2bc)

    w_node = w3d.transpose(0, 2, 1).reshape(3, EPAD)
    idx_flat = nbr_idx.reshape(-1)
    idx_pad = jnp.pad(idx_flat, (0, EPAD - n * m)).reshape(NCHUNKS, IDX_ROWS, 128)
    w_chunks = [w_node[l].reshape(NCHUNKS, CHUNK_EDGES) for l in range(3)]

    t = _make_agg_kernel(True)(x0, idx_pad, w_chunks[0])
    agg_i = _make_agg_kernel(False)
    t = agg_i(t, idx_pad, w_chunks[1])
    t = agg_i(t, idx_pad, w_chunks[2])

    cry_idx = crystal_atom_idx.reshape(NW, (n_cry * ap) // (NW * 128), 128)
    pool = _make_pool_kernel(n_cry, ap)(t, cry_idx)

    out, h = _head_call(pool, W_tp_0, W_tp_1, W_tp_2, W_fc,
                        b_fc.reshape(1, -1), W_out, b_out.reshape(1, 1))
    return (out, h)


# speedup vs baseline: 1.2828x; 1.0756x over previous
import functools
import math

import jax
import jax.numpy as jnp
from jax import lax
from jax.experimental import pallas as pl
from jax.experimental.pallas import tpu as pltpu
from jax.experimental.pallas import tpu_sc as plsc

N = 50000
M = 16
NBR = 41
A = 64
LANES = 16

NW = 32
CHUNK_NODES = 32
CHUNK_EDGES = CHUNK_NODES * M
IDX_ROWS = CHUNK_EDGES // 128
CHUNKS_PER_WORKER = 49
NPW = CHUNKS_PER_WORKER * CHUNK_NODES
NPAD = NPW * NW
EPAD = NPAD * M
NCHUNKS = NPAD // CHUNK_NODES

BN = 512
BE = BN * M
GRID1 = NPAD // BN

PAIR_CHUNKS = 2 * CHUNKS_PER_WORKER
K0 = 49
MAXP = (max(K0, PAIR_CHUNKS - K0) + 1) // 2


A32 = 32


def _unpack_bf16(u):
    lo = plsc.bitcast(u << 16, jnp.float32)
    hi = plsc.bitcast(u & jnp.int32(-65536), jnp.float32)
    return lo, hi


def _pack_bf16(lo, hi):
    ve = plsc.bitcast(lo, jnp.int32)
    re = ve + (jnp.int32(0x7FFF) + ((ve >> 16) & 1))
    vo = plsc.bitcast(hi, jnp.int32)
    ro = vo + (jnp.int32(0x7FFF) + ((vo >> 16) & 1))
    return lax.shift_right_logical(re, 16) | (ro & jnp.int32(-65536))


def _bcast_lane(v, j):
    return lax.gather(
        v, jnp.full((LANES, 1), j, jnp.int32),
        lax.GatherDimensionNumbers(
            offset_dims=(), collapsed_slice_dims=(0,), start_index_map=(0,)),
        slice_sizes=(1,), mode=lax.GatherScatterMode.PROMISE_IN_BOUNDS)


def _softplus(x):
    return jnp.maximum(x, 0.0) + jnp.log1p(jnp.exp(-jnp.abs(x)))


def _tc_pack_bf16(a, b):
    va = lax.bitcast_convert_type(a, jnp.int32)
    ra = va + (jnp.int32(0x7FFF) + ((va >> 16) & 1))
    vb = lax.bitcast_convert_type(b, jnp.int32)
    rb = vb + (jnp.int32(0x7FFF) + ((vb >> 16) & 1))
    return lax.shift_right_logical(ra, 16) | (rb & jnp.int32(-65536))


def _prep_body(atom_t, nbr_t, wemb, bemb, wr1cat, br1cat, w2sel, b2bc, x0_o, w_o):
    x = lax.dot_general(atom_t[...], wemb[...], (((0,), (0,)), ((), ())),
                        preferred_element_type=jnp.float32) + bemb[...]
    for h in range(A // 32):
        x0_o[:, h * LANES:(h + 1) * LANES] = _tc_pack_bf16(
            x[:, 32 * h:32 * h + LANES], x[:, 32 * h + LANES:32 * h + 2 * LANES])
    nb = nbr_t[...].reshape(NBR, BE)
    z = lax.dot_general(nb, wr1cat[...], (((0,), (0,)), ((), ())),
                        preferred_element_type=jnp.float32) + br1cat[...]
    s = jnp.log1p(jnp.exp(z))
    y3 = lax.dot_general(w2sel[...], s, (((0,), (1,)), ((), ())),
                         preferred_element_type=jnp.float32)
    scale = 1.0 / (M * math.sqrt(A))
    w_o[...] = ((y3 + b2bc[...]) * scale).reshape(3, M, BN)


def _prep_call(atom_t, nbr_t3, W_emb, b_emb2, wr1cat, br1cat, w2sel, b2bc):
    return pl.pallas_call(
        _prep_body,
        grid=(GRID1,),
        in_specs=[
            pl.BlockSpec((atom_t.shape[0], BN), lambda i: (0, i)),
            pl.BlockSpec((NBR, M, BN), lambda i: (0, 0, i)),
            pl.BlockSpec(W_emb.shape, lambda i: (0, 0)),
            pl.BlockSpec(b_emb2.shape, lambda i: (0, 0)),
            pl.BlockSpec(wr1cat.shape, lambda i: (0, 0)),
            pl.BlockSpec(br1cat.shape, lambda i: (0, 0)),
            pl.BlockSpec(w2sel.shape, lambda i: (0, 0)),
            pl.BlockSpec((3, BE), lambda i: (0, 0)),
        ],
        out_specs=[
            pl.BlockSpec((BN, A32), lambda i: (i, 0)),
            pl.BlockSpec((3, M, BN), lambda i: (0, 0, i)),
        ],
        out_shape=[
            jax.ShapeDtypeStruct((NPAD, A32), jnp.int32),
            jax.ShapeDtypeStruct((3, M, NPAD), jnp.float32),
        ],
    )(atom_t, nbr_t3, W_emb, b_emb2, wr1cat, br1cat, w2sel, b2bc)


def _make_agg_kernel(f32_in):
    mesh = plsc.VectorSubcoreMesh(core_axis_name="c", subcore_axis_name="s",
                                  num_cores=2, num_subcores=16)
    row_w = A if f32_in else A32

    def body(table_h, idx_h, w_h, out_h,
             idx_v, w_v, rows_v, out_v,
             sg0, sg1, si0, si1, sw0, sw1, so0, so1):
        cid = lax.axis_index("c")
        sid = lax.axis_index("s")
        n = jnp.where(cid == 0, K0, PAIR_CHUNKS - K0)
        chunk0 = sid * PAIR_CHUNKS + jnp.where(cid == 0, 0, K0)

        sg = (sg0, sg1)
        si = (si0, si1)
        sw = (sw0, sw1)
        so = (so0, so1)

        def fire_gathers(b):
            for r in range(IDX_ROWS):
                pltpu.async_copy(
                    table_h.at[idx_v.at[b, r]],
                    rows_v.at[b, pl.ds(r * 128, 128)],
                    sg[b])

        def drain_gathers(b):
            for r in range(IDX_ROWS):
                pltpu.make_async_copy(
                    table_h.at[idx_v.at[b, r]],
                    rows_v.at[b, pl.ds(r * 128, 128)],
                    sg[b]).wait()

        for b in range(2):
            pltpu.sync_copy(idx_h.at[chunk0 + b], idx_v.at[b])
            fire_gathers(b)
            pltpu.async_copy(w_h.at[chunk0 + b], w_v.at[b], sw[b])

        def compute_chunk(b, cg):
            def node_body(nl, _):
                base = nl * M
                wrow = w_v[b, pl.ds(base, LANES)]
                acc = [jnp.zeros((LANES,), jnp.float32) for _ in range(A // LANES)]
                for j in range(M):
                    e = base + j
                    wj = _bcast_lane(wrow, j)
                    if f32_in:
                        for q in range(A // LANES):
                            acc[q] = acc[q] + wj * rows_v[b, e, pl.ds(q * LANES, LANES)]
                    else:
                        for h in range(A32 // LANES):
                            lo, hi = _unpack_bf16(rows_v[b, e, pl.ds(h * LANES, LANES)])
                            acc[2 * h] = acc[2 * h] + wj * lo
                            acc[2 * h + 1] = acc[2 * h + 1] + wj * hi
                for h in range(A32 // LANES):
                    out_v[b, nl, pl.ds(h * LANES, LANES)] = _pack_bf16(
                        acc[2 * h], acc[2 * h + 1])
                return 0
            lax.fori_loop(0, CHUNK_NODES, node_body, 0)
            pltpu.async_copy(
                out_v.at[b],
                out_h.at[pl.ds(cg * CHUNK_NODES, CHUNK_NODES)],
                so[b])

        def wait_out(b, cg):
            pltpu.make_async_copy(
                out_v.at[b],
                out_h.at[pl.ds(cg * CHUNK_NODES, CHUNK_NODES)],
                so[b]).wait()

        def outer(it, carry):
            for b in range(2):
                c = it * 2 + b
                cg = chunk0 + c

                @pl.when(c < n)
                def _():
                    drain_gathers(b)

                    @pl.when(c + 2 < n)
                    def _():
                        pltpu.async_copy(idx_h.at[cg + 2], idx_v.at[b], si[b])

                    pltpu.make_async_copy(w_h.at[cg], w_v.at[b], sw[b]).wait()

                    @pl.when(c >= 2)
                    def _():
                        wait_out(b, cg - 2)

                    compute_chunk(b, cg)

                    @pl.when(c + 2 < n)
                    def _():
                        pltpu.make_async_copy(
                            idx_h.at[cg + 2], idx_v.at[b], si[b]).wait()
                        fire_gathers(b)
                        pltpu.async_copy(w_h.at[cg + 2], w_v.at[b], sw[b])

            return carry

        lax.fori_loop(0, MAXP, outer, 0)
        wait_out(0, chunk0)
        wait_out(1, chunk0)

    kern = pl.kernel(
        body,
        out_type=jax.ShapeDtypeStruct((NPAD, A32), jnp.int32),
        mesh=mesh,
        scratch_types=[
            pltpu.VMEM((2, IDX_ROWS, 128), jnp.int32),
            pltpu.VMEM((2, CHUNK_EDGES), jnp.float32),
            pltpu.VMEM((2, CHUNK_EDGES, row_w),
                       jnp.float32 if f32_in else jnp.int32),
            pltpu.VMEM((2, CHUNK_NODES, A32), jnp.int32),
        ] + [pltpu.SemaphoreType.DMA] * 8,
        compiler_params=pltpu.CompilerParams(use_tc_tiling_on_sc=False,
                                             needs_layout_passes=False),
    )
    return kern


def _make_pool_kernel(n_cry, ap):
    mesh = plsc.VectorSubcoreMesh(core_axis_name="c", subcore_axis_name="s",
                                  num_cores=2, num_subcores=16)
    cpw = n_cry // NW
    ipw = cpw * ap
    rows128 = ipw // 128

    def body(table_h, idx_h, out_h, idx_v, rows_v, out_v, sg):
        cid = lax.axis_index("c")
        sid = lax.axis_index("s")
        wid = sid * 2 + cid
        pltpu.sync_copy(idx_h.at[wid], idx_v)
        for r in range(rows128):
            pltpu.async_copy(
                table_h.at[idx_v.at[r]],
                rows_v.at[pl.ds(r * 128, 128)], sg)
        for r in range(rows128):
            pltpu.make_async_copy(
                table_h.at[idx_v.at[r]],
                rows_v.at[pl.ds(r * 128, 128)], sg).wait()
        inv = 1.0 / ap

        def cry_body(k, _):
            base = k * ap
            acc = [jnp.zeros((LANES,), jnp.float32) for _ in range(2 * (A32 // LANES))]
            for j in range(ap):
                for h in range(A32 // LANES):
                    lo, hi = _unpack_bf16(rows_v[base + j, pl.ds(h * LANES, LANES)])
                    acc[2 * h] = acc[2 * h] + lo
                    acc[2 * h + 1] = acc[2 * h + 1] + hi
            for i in range(A // LANES):
                out_v[k, pl.ds(i * LANES, LANES)] = acc[i] * inv
            return 0
        lax.fori_loop(0, cpw, cry_body, 0)
        pltpu.sync_copy(out_v, out_h.at[pl.ds(wid * cpw, cpw)])

    kern = pl.kernel(
        body,
        out_type=jax.ShapeDtypeStruct((n_cry, A), jnp.float32),
        mesh=mesh,
        scratch_types=[
            pltpu.VMEM((rows128, 128), jnp.int32),
            pltpu.VMEM((ipw, A32), jnp.int32),
            pltpu.VMEM((cpw, A), jnp.float32),
            pltpu.SemaphoreType.DMA,
        ],
        compiler_params=pltpu.CompilerParams(use_tc_tiling_on_sc=False,
                                             needs_layout_passes=False),
    )
    return kern


def _head_body(pool, wt0, wt1, wt2, wfc, bfc, wout, bout, out_o, h_o):
    wc = jnp.dot(jnp.dot(wt0[...], wt1[...], preferred_element_type=jnp.float32),
                 wt2[...], preferred_element_type=jnp.float32)
    crys = jnp.dot(pool[...].astype(jnp.float32), wc,
                   preferred_element_type=jnp.float32)
    pre = jnp.dot(crys, wfc[...], preferred_element_type=jnp.float32) + bfc[...]
    h = _softplus(pre)
    h_o[...] = h
    out_o[...] = jnp.dot(h, wout[...], preferred_element_type=jnp.float32) + bout[...]


def _head_call(pool, W_tp_0, W_tp_1, W_tp_2, W_fc, b_fc2, W_out, b_out2):
    n_cry = pool.shape[0]
    H = W_fc.shape[1]
    return pl.pallas_call(
        _head_body,
        out_shape=[
            jax.ShapeDtypeStruct((n_cry, 1), jnp.float32),
            jax.ShapeDtypeStruct((n_cry, H), jnp.float32),
        ],
    )(pool, W_tp_0, W_tp_1, W_tp_2, W_fc, b_fc2, W_out, b_out2)


def kernel(atom_fea, nbr_fea, nbr_idx, crystal_atom_idx, W_emb, b_emb,
           W_r1_0, b_r1_0, W_r2_0, b_r2_0, W_tp_0,
           W_r1_1, b_r1_1, W_r2_1, b_r2_1, W_tp_1,
           W_r1_2, b_r1_2, W_r2_2, b_r2_2, W_tp_2,
           W_fc, b_fc, W_out, b_out):
    n, m = nbr_idx.shape
    n_cry, ap = crystal_atom_idx.shape

    atom_t = atom_fea.T
    nbr_t3 = jnp.transpose(nbr_fea, (2, 1, 0))
    wr1cat = jnp.concatenate([W_r1_0, W_r1_1, W_r1_2], axis=1)
    br1cat = jnp.concatenate([b_r1_0, b_r1_1, b_r1_2]).reshape(1, 3 * NBR)
    w2sel = jnp.zeros((3 * NBR, 3), jnp.float32)
    w2sel = w2sel.at[0 * NBR:1 * NBR, 0].set(W_r2_0[:, 0])
    w2sel = w2sel.at[1 * NBR:2 * NBR, 1].set(W_r2_1[:, 0])
    w2sel = w2sel.at[2 * NBR:3 * NBR, 2].set(W_r2_2[:, 0])
    b2bc = jnp.broadcast_to(
        jnp.stack([b_r2_0[0], b_r2_1[0], b_r2_2[0]]).reshape(3, 1), (3, BE))
    b_emb2 = b_emb.reshape(1, A)

    x0, w3d = _prep_call(atom_t, nbr_t3, W_emb, b_emb2, wr1cat, br1cat, w2sel, b2bc)

    w_node = w3d.transpose(0, 2, 1).reshape(3, EPAD)
    idx_flat = nbr_idx.reshape(-1)
    idx_pad = jnp.pad(idx_flat, (0, EPAD - n * m)).reshape(NCHUNKS, IDX_ROWS, 128)
    w_chunks = [w_node[l].reshape(NCHUNKS, CHUNK_EDGES) for l in range(3)]

    agg_i = _make_agg_kernel(False)
    t = x0
    for l in range(3):
        t = agg_i(t, idx_pad, w_chunks[l])

    cry_idx = crystal_atom_idx.reshape(NW, (n_cry * ap) // (NW * 128), 128)
    pool = _make_pool_kernel(n_cry, ap)(t, cry_idx)

    out, h = _head_call(pool, W_tp_0, W_tp_1, W_tp_2, W_fc,
                        b_fc.reshape(1, -1), W_out, b_out.reshape(1, 1))
    return (out, h)


# speedup vs baseline: 1.3217x; 1.0304x over previous
import functools
import math

import jax
import jax.numpy as jnp
from jax import lax
from jax.experimental import pallas as pl
from jax.experimental.pallas import tpu as pltpu
from jax.experimental.pallas import tpu_sc as plsc

N = 50000
M = 16
NBR = 41
A = 64
LANES = 16

NW = 32
CHUNK_NODES = 32
CHUNK_EDGES = CHUNK_NODES * M
IDX_ROWS = CHUNK_EDGES // 128
CHUNKS_PER_WORKER = 49
NPW = CHUNKS_PER_WORKER * CHUNK_NODES
NPAD = NPW * NW
EPAD = NPAD * M
NCHUNKS = NPAD // CHUNK_NODES

BN = 512
BE = BN * M
GRID1 = NPAD // BN

PAIR_CHUNKS = 2 * CHUNKS_PER_WORKER
K0 = 57
MAXP = (max(K0, PAIR_CHUNKS - K0) + 1) // 2


A32 = 32


def _unpack_bf16(u):
    lo = plsc.bitcast(u << 16, jnp.float32)
    hi = plsc.bitcast(u & jnp.int32(-65536), jnp.float32)
    return lo, hi


def _pack_bf16(lo, hi):
    ve = plsc.bitcast(lo, jnp.int32)
    re = ve + (jnp.int32(0x7FFF) + ((ve >> 16) & 1))
    vo = plsc.bitcast(hi, jnp.int32)
    ro = vo + (jnp.int32(0x7FFF) + ((vo >> 16) & 1))
    return lax.shift_right_logical(re, 16) | (ro & jnp.int32(-65536))


def _bcast_lane(v, j):
    return lax.gather(
        v, jnp.full((LANES, 1), j, jnp.int32),
        lax.GatherDimensionNumbers(
            offset_dims=(), collapsed_slice_dims=(0,), start_index_map=(0,)),
        slice_sizes=(1,), mode=lax.GatherScatterMode.PROMISE_IN_BOUNDS)


def _softplus(x):
    return jnp.maximum(x, 0.0) + jnp.log1p(jnp.exp(-jnp.abs(x)))


def _tc_pack_bf16(a, b):
    va = lax.bitcast_convert_type(a, jnp.int32)
    ra = va + (jnp.int32(0x7FFF) + ((va >> 16) & 1))
    vb = lax.bitcast_convert_type(b, jnp.int32)
    rb = vb + (jnp.int32(0x7FFF) + ((vb >> 16) & 1))
    return lax.shift_right_logical(ra, 16) | (rb & jnp.int32(-65536))


def _prep_body(atom_t, nbr_t, wemb, bemb, wr1cat, br1cat, w2sel, b2bc, x0_o, w_o):
    x = lax.dot_general(atom_t[...], wemb[...], (((0,), (0,)), ((), ())),
                        preferred_element_type=jnp.float32) + bemb[...]
    for h in range(A // 32):
        x0_o[:, h * LANES:(h + 1) * LANES] = _tc_pack_bf16(
            x[:, 32 * h:32 * h + LANES], x[:, 32 * h + LANES:32 * h + 2 * LANES])
    nb = nbr_t[...].reshape(NBR, BE).astype(jnp.bfloat16)
    z = lax.dot_general(nb, wr1cat[...].astype(jnp.bfloat16),
                        (((0,), (0,)), ((), ())),
                        preferred_element_type=jnp.float32) + br1cat[...]
    s = jnp.log1p(jnp.exp(z))
    y3 = lax.dot_general(w2sel[...], s, (((0,), (1,)), ((), ())),
                         preferred_element_type=jnp.float32)
    scale = 1.0 / (M * math.sqrt(A))
    w_o[...] = ((y3 + b2bc[...]) * scale).reshape(3, M, BN)


def _prep_call(atom_t, nbr_t3, W_emb, b_emb2, wr1cat, br1cat, w2sel, b2bc):
    return pl.pallas_call(
        _prep_body,
        grid=(GRID1,),
        in_specs=[
            pl.BlockSpec((atom_t.shape[0], BN), lambda i: (0, i)),
            pl.BlockSpec((NBR, M, BN), lambda i: (0, 0, i)),
            pl.BlockSpec(W_emb.shape, lambda i: (0, 0)),
            pl.BlockSpec(b_emb2.shape, lambda i: (0, 0)),
            pl.BlockSpec(wr1cat.shape, lambda i: (0, 0)),
            pl.BlockSpec(br1cat.shape, lambda i: (0, 0)),
            pl.BlockSpec(w2sel.shape, lambda i: (0, 0)),
            pl.BlockSpec((3, BE), lambda i: (0, 0)),
        ],
        out_specs=[
            pl.BlockSpec((BN, A32), lambda i: (i, 0)),
            pl.BlockSpec((3, M, BN), lambda i: (0, 0, i)),
        ],
        out_shape=[
            jax.ShapeDtypeStruct((NPAD, A32), jnp.int32),
            jax.ShapeDtypeStruct((3, M, NPAD), jnp.float32),
        ],
    )(atom_t, nbr_t3, W_emb, b_emb2, wr1cat, br1cat, w2sel, b2bc)


def _make_agg_kernel(f32_in):
    mesh = plsc.VectorSubcoreMesh(core_axis_name="c", subcore_axis_name="s",
                                  num_cores=2, num_subcores=16)
    row_w = A if f32_in else A32

    def body(table_h, idx_h, w_h, out_h,
             idx_v, w_v, rows_v, out_v,
             sg0, sg1, si0, si1, sw0, sw1, so0, so1):
        cid = lax.axis_index("c")
        sid = lax.axis_index("s")
        n = jnp.where(cid == 0, K0, PAIR_CHUNKS - K0)
        chunk0 = sid * PAIR_CHUNKS + jnp.where(cid == 0, 0, K0)

        sg = (sg0, sg1)
        si = (si0, si1)
        sw = (sw0, sw1)
        so = (so0, so1)

        def fire_gathers(b):
            for r in range(IDX_ROWS):
                pltpu.async_copy(
                    table_h.at[idx_v.at[b, r]],
                    rows_v.at[b, pl.ds(r * 128, 128)],
                    sg[b])

        def drain_gathers(b):
            for r in range(IDX_ROWS):
                pltpu.make_async_copy(
                    table_h.at[idx_v.at[b, r]],
                    rows_v.at[b, pl.ds(r * 128, 128)],
                    sg[b]).wait()

        for b in range(2):
            pltpu.sync_copy(idx_h.at[chunk0 + b], idx_v.at[b])
            fire_gathers(b)
            pltpu.async_copy(w_h.at[chunk0 + b], w_v.at[b], sw[b])

        def compute_chunk(b, cg):
            def node_body(nl, _):
                base = nl * M
                wrow = w_v[b, pl.ds(base, LANES)]
                acc = [jnp.zeros((LANES,), jnp.float32) for _ in range(A // LANES)]
                for j in range(M):
                    e = base + j
                    wj = _bcast_lane(wrow, j)
                    if f32_in:
                        for q in range(A // LANES):
                            acc[q] = acc[q] + wj * rows_v[b, e, pl.ds(q * LANES, LANES)]
                    else:
                        for h in range(A32 // LANES):
                            lo, hi = _unpack_bf16(rows_v[b, e, pl.ds(h * LANES, LANES)])
                            acc[2 * h] = acc[2 * h] + wj * lo
                            acc[2 * h + 1] = acc[2 * h + 1] + wj * hi
                for h in range(A32 // LANES):
                    out_v[b, nl, pl.ds(h * LANES, LANES)] = _pack_bf16(
                        acc[2 * h], acc[2 * h + 1])
                return 0
            lax.fori_loop(0, CHUNK_NODES, node_body, 0)
            pltpu.async_copy(
                out_v.at[b],
                out_h.at[pl.ds(cg * CHUNK_NODES, CHUNK_NODES)],
                so[b])

        def wait_out(b, cg):
            pltpu.make_async_copy(
                out_v.at[b],
                out_h.at[pl.ds(cg * CHUNK_NODES, CHUNK_NODES)],
                so[b]).wait()

        def outer(it, carry):
            for b in range(2):
                c = it * 2 + b
                cg = chunk0 + c

                @pl.when(c < n)
                def _():
                    drain_gathers(b)

                    @pl.when(c + 2 < n)
                    def _():
                        pltpu.async_copy(idx_h.at[cg + 2], idx_v.at[b], si[b])

                    pltpu.make_async_copy(w_h.at[cg], w_v.at[b], sw[b]).wait()

                    @pl.when(c >= 2)
                    def _():
                        wait_out(b, cg - 2)

                    compute_chunk(b, cg)

                    @pl.when(c + 2 < n)
                    def _():
                        pltpu.make_async_copy(
                            idx_h.at[cg + 2], idx_v.at[b], si[b]).wait()
                        fire_gathers(b)
                        pltpu.async_copy(w_h.at[cg + 2], w_v.at[b], sw[b])

            return carry

        lax.fori_loop(0, MAXP, outer, 0)
        wait_out(0, chunk0)
        wait_out(1, chunk0)

    kern = pl.kernel(
        body,
        out_type=jax.ShapeDtypeStruct((NPAD, A32), jnp.int32),
        mesh=mesh,
        scratch_types=[
            pltpu.VMEM((2, IDX_ROWS, 128), jnp.int32),
            pltpu.VMEM((2, CHUNK_EDGES), jnp.float32),
            pltpu.VMEM((2, CHUNK_EDGES, row_w),
                       jnp.float32 if f32_in else jnp.int32),
            pltpu.VMEM((2, CHUNK_NODES, A32), jnp.int32),
        ] + [pltpu.SemaphoreType.DMA] * 8,
        compiler_params=pltpu.CompilerParams(use_tc_tiling_on_sc=False,
                                             needs_layout_passes=False),
    )
    return kern


def _make_pool_kernel(n_cry, ap):
    mesh = plsc.VectorSubcoreMesh(core_axis_name="c", subcore_axis_name="s",
                                  num_cores=2, num_subcores=16)
    cpw = n_cry // NW
    ipw = cpw * ap
    rows128 = ipw // 128

    def body(table_h, idx_h, out_h, idx_v, rows_v, out_v, sg):
        cid = lax.axis_index("c")
        sid = lax.axis_index("s")
        wid = sid * 2 + cid
        pltpu.sync_copy(idx_h.at[wid], idx_v)
        for r in range(rows128):
            pltpu.async_copy(
                table_h.at[idx_v.at[r]],
                rows_v.at[pl.ds(r * 128, 128)], sg)
        for r in range(rows128):
            pltpu.make_async_copy(
                table_h.at[idx_v.at[r]],
                rows_v.at[pl.ds(r * 128, 128)], sg).wait()
        inv = 1.0 / ap

        def cry_body(k, _):
            base = k * ap
            acc = [jnp.zeros((LANES,), jnp.float32) for _ in range(2 * (A32 // LANES))]
            for j in range(ap):
                for h in range(A32 // LANES):
                    lo, hi = _unpack_bf16(rows_v[base + j, pl.ds(h * LANES, LANES)])
                    acc[2 * h] = acc[2 * h] + lo
                    acc[2 * h + 1] = acc[2 * h + 1] + hi
            for i in range(A // LANES):
                out_v[k, pl.ds(i * LANES, LANES)] = acc[i] * inv
            return 0
        lax.fori_loop(0, cpw, cry_body, 0)
        pltpu.sync_copy(out_v, out_h.at[pl.ds(wid * cpw, cpw)])

    kern = pl.kernel(
        body,
        out_type=jax.ShapeDtypeStruct((n_cry, A), jnp.float32),
        mesh=mesh,
        scratch_types=[
            pltpu.VMEM((rows128, 128), jnp.int32),
            pltpu.VMEM((ipw, A32), jnp.int32),
            pltpu.VMEM((cpw, A), jnp.float32),
            pltpu.SemaphoreType.DMA,
        ],
        compiler_params=pltpu.CompilerParams(use_tc_tiling_on_sc=False,
                                             needs_layout_passes=False),
    )
    return kern


def _head_body(pool, wt0, wt1, wt2, wfc, bfc, wout, bout, out_o, h_o):
    wc = jnp.dot(jnp.dot(wt0[...], wt1[...], preferred_element_type=jnp.float32),
                 wt2[...], preferred_element_type=jnp.float32)
    crys = jnp.dot(pool[...].astype(jnp.float32), wc,
                   preferred_element_type=jnp.float32)
    pre = jnp.dot(crys, wfc[...], preferred_element_type=jnp.float32) + bfc[...]
    h = _softplus(pre)
    h_o[...] = h
    out_o[...] = jnp.dot(h, wout[...], preferred_element_type=jnp.float32) + bout[...]


def _head_call(pool, W_tp_0, W_tp_1, W_tp_2, W_fc, b_fc2, W_out, b_out2):
    n_cry = pool.shape[0]
    H = W_fc.shape[1]
    return pl.pallas_call(
        _head_body,
        out_shape=[
            jax.ShapeDtypeStruct((n_cry, 1), jnp.float32),
            jax.ShapeDtypeStruct((n_cry, H), jnp.float32),
        ],
    )(pool, W_tp_0, W_tp_1, W_tp_2, W_fc, b_fc2, W_out, b_out2)


def kernel(atom_fea, nbr_fea, nbr_idx, crystal_atom_idx, W_emb, b_emb,
           W_r1_0, b_r1_0, W_r2_0, b_r2_0, W_tp_0,
           W_r1_1, b_r1_1, W_r2_1, b_r2_1, W_tp_1,
           W_r1_2, b_r1_2, W_r2_2, b_r2_2, W_tp_2,
           W_fc, b_fc, W_out, b_out):
    n, m = nbr_idx.shape
    n_cry, ap = crystal_atom_idx.shape

    atom_t = atom_fea.T
    nbr_t3 = jnp.transpose(nbr_fea, (2, 1, 0))
    wr1cat = jnp.concatenate([W_r1_0, W_r1_1, W_r1_2], axis=1)
    br1cat = jnp.concatenate([b_r1_0, b_r1_1, b_r1_2]).reshape(1, 3 * NBR)
    w2sel = jnp.zeros((3 * NBR, 3), jnp.float32)
    w2sel = w2sel.at[0 * NBR:1 * NBR, 0].set(W_r2_0[:, 0])
    w2sel = w2sel.at[1 * NBR:2 * NBR, 1].set(W_r2_1[:, 0])
    w2sel = w2sel.at[2 * NBR:3 * NBR, 2].set(W_r2_2[:, 0])
    b2bc = jnp.broadcast_to(
        jnp.stack([b_r2_0[0], b_r2_1[0], b_r2_2[0]]).reshape(3, 1), (3, BE))
    b_emb2 = b_emb.reshape(1, A)

    x0, w3d = _prep_call(atom_t, nbr_t3, W_emb, b_emb2, wr1cat, br1cat, w2sel, b2bc)

    w_node = w3d.transpose(0, 2, 1).reshape(3, EPAD)
    idx_flat = nbr_idx.reshape(-1)
    idx_pad = jnp.pad(idx_flat, (0, EPAD - n * m)).reshape(NCHUNKS, IDX_ROWS, 128)
    w_chunks = [w_node[l].reshape(NCHUNKS, CHUNK_EDGES) for l in range(3)]

    agg_i = _make_agg_kernel(False)
    t = x0
    for l in range(3):
        t = agg_i(t, idx_pad, w_chunks[l])

    cry_idx = crystal_atom_idx.reshape(NW, (n_cry * ap) // (NW * 128), 128)
    pool = _make_pool_kernel(n_cry, ap)(t, cry_idx)

    out, h = _head_call(pool, W_tp_0, W_tp_1, W_tp_2, W_fc,
                        b_fc.reshape(1, -1), W_out, b_out.reshape(1, 1))
    return (out, h)


# speedup vs baseline: 1.3348x; 1.0099x over previous
import functools
import math

import jax
import jax.numpy as jnp
from jax import lax
from jax.experimental import pallas as pl
from jax.experimental.pallas import tpu as pltpu
from jax.experimental.pallas import tpu_sc as plsc

N = 50000
M = 16
NBR = 41
A = 64
LANES = 16

NW = 32
CHUNK_NODES = 32
CHUNK_EDGES = CHUNK_NODES * M
IDX_ROWS = CHUNK_EDGES // 128
CHUNKS_PER_WORKER = 49
NPW = CHUNKS_PER_WORKER * CHUNK_NODES
NPAD = NPW * NW
EPAD = NPAD * M
NCHUNKS = NPAD // CHUNK_NODES

BN = 1024
BE = BN * M
GRID1 = NPAD // BN

PAIR_CHUNKS = 2 * CHUNKS_PER_WORKER
K0 = 57
MAXP = (max(K0, PAIR_CHUNKS - K0) + 1) // 2


A32 = 32


def _unpack_bf16(u):
    lo = plsc.bitcast(u << 16, jnp.float32)
    hi = plsc.bitcast(u & jnp.int32(-65536), jnp.float32)
    return lo, hi


def _pack_bf16(lo, hi):
    ve = plsc.bitcast(lo, jnp.int32)
    re = ve + (jnp.int32(0x7FFF) + ((ve >> 16) & 1))
    vo = plsc.bitcast(hi, jnp.int32)
    ro = vo + (jnp.int32(0x7FFF) + ((vo >> 16) & 1))
    return lax.shift_right_logical(re, 16) | (ro & jnp.int32(-65536))


def _bcast_lane(v, j):
    return lax.gather(
        v, jnp.full((LANES, 1), j, jnp.int32),
        lax.GatherDimensionNumbers(
            offset_dims=(), collapsed_slice_dims=(0,), start_index_map=(0,)),
        slice_sizes=(1,), mode=lax.GatherScatterMode.PROMISE_IN_BOUNDS)


def _softplus(x):
    return jnp.maximum(x, 0.0) + jnp.log1p(jnp.exp(-jnp.abs(x)))


def _tc_pack_bf16(a, b):
    va = lax.bitcast_convert_type(a, jnp.int32)
    ra = va + (jnp.int32(0x7FFF) + ((va >> 16) & 1))
    vb = lax.bitcast_convert_type(b, jnp.int32)
    rb = vb + (jnp.int32(0x7FFF) + ((vb >> 16) & 1))
    return lax.shift_right_logical(ra, 16) | (rb & jnp.int32(-65536))


def _prep_body(atom_t, nbr_t, wemb, bemb, wr1cat, br1cat, w2sel, b2bc, x0_o, w_o):
    x = lax.dot_general(atom_t[...], wemb[...], (((0,), (0,)), ((), ())),
                        preferred_element_type=jnp.float32) + bemb[...]
    for h in range(A // 32):
        x0_o[:, h * LANES:(h + 1) * LANES] = _tc_pack_bf16(
            x[:, 32 * h:32 * h + LANES], x[:, 32 * h + LANES:32 * h + 2 * LANES])
    nb = nbr_t[...].reshape(NBR, BE)
    z = lax.dot_general(nb, wr1cat[...], (((0,), (0,)), ((), ())),
                        preferred_element_type=jnp.float32) + br1cat[...]
    s = jnp.log1p(jnp.exp(z))
    y3 = lax.dot_general(w2sel[...], s, (((0,), (1,)), ((), ())),
                         preferred_element_type=jnp.float32)
    scale = 1.0 / (M * math.sqrt(A))
    w_o[...] = ((y3 + b2bc[...]) * scale).reshape(3, M, BN)


def _prep_call(atom_t, nbr_t3, W_emb, b_emb2, wr1cat, br1cat, w2sel, b2bc):
    return pl.pallas_call(
        _prep_body,
        grid=(GRID1,),
        in_specs=[
            pl.BlockSpec((atom_t.shape[0], BN), lambda i: (0, i)),
            pl.BlockSpec((NBR, M, BN), lambda i: (0, 0, i)),
            pl.BlockSpec(W_emb.shape, lambda i: (0, 0)),
            pl.BlockSpec(b_emb2.shape, lambda i: (0, 0)),
            pl.BlockSpec(wr1cat.shape, lambda i: (0, 0)),
            pl.BlockSpec(br1cat.shape, lambda i: (0, 0)),
            pl.BlockSpec(w2sel.shape, lambda i: (0, 0)),
            pl.BlockSpec((3, BE), lambda i: (0, 0)),
        ],
        out_specs=[
            pl.BlockSpec((BN, A32), lambda i: (i, 0)),
            pl.BlockSpec((3, M, BN), lambda i: (0, 0, i)),
        ],
        out_shape=[
            jax.ShapeDtypeStruct((NPAD, A32), jnp.int32),
            jax.ShapeDtypeStruct((3, M, NPAD), jnp.float32),
        ],
    )(atom_t, nbr_t3, W_emb, b_emb2, wr1cat, br1cat, w2sel, b2bc)


def _make_agg_kernel(f32_in):
    mesh = plsc.VectorSubcoreMesh(core_axis_name="c", subcore_axis_name="s",
                                  num_cores=2, num_subcores=16)
    row_w = A if f32_in else A32

    def body(table_h, idx_h, w_h, out_h,
             idx_v, w_v, rows_v, out_v,
             sg0, sg1, si0, si1, sw0, sw1, so0, so1):
        cid = lax.axis_index("c")
        sid = lax.axis_index("s")
        n = jnp.where(cid == 0, K0, PAIR_CHUNKS - K0)
        chunk0 = sid * PAIR_CHUNKS + jnp.where(cid == 0, 0, K0)

        sg = (sg0, sg1)
        si = (si0, si1)
        sw = (sw0, sw1)
        so = (so0, so1)

        def fire_gathers(b):
            for r in range(IDX_ROWS):
                pltpu.async_copy(
                    table_h.at[idx_v.at[b, r]],
                    rows_v.at[b, pl.ds(r * 128, 128)],
                    sg[b])

        def drain_gathers(b):
            for r in range(IDX_ROWS):
                pltpu.make_async_copy(
                    table_h.at[idx_v.at[b, r]],
                    rows_v.at[b, pl.ds(r * 128, 128)],
                    sg[b]).wait()

        for b in range(2):
            pltpu.sync_copy(idx_h.at[chunk0 + b], idx_v.at[b])
            fire_gathers(b)
            pltpu.async_copy(w_h.at[chunk0 + b], w_v.at[b], sw[b])

        def compute_chunk(b, cg):
            def node_body(nl, _):
                base = nl * M
                wrow = w_v[b, pl.ds(base, LANES)]
                acc = [jnp.zeros((LANES,), jnp.float32) for _ in range(A // LANES)]
                for j in range(M):
                    e = base + j
                    wj = _bcast_lane(wrow, j)
                    if f32_in:
                        for q in range(A // LANES):
                            acc[q] = acc[q] + wj * rows_v[b, e, pl.ds(q * LANES, LANES)]
                    else:
                        for h in range(A32 // LANES):
                            lo, hi = _unpack_bf16(rows_v[b, e, pl.ds(h * LANES, LANES)])
                            acc[2 * h] = acc[2 * h] + wj * lo
                            acc[2 * h + 1] = acc[2 * h + 1] + wj * hi
                for h in range(A32 // LANES):
                    out_v[b, nl, pl.ds(h * LANES, LANES)] = _pack_bf16(
                        acc[2 * h], acc[2 * h + 1])
                return 0
            lax.fori_loop(0, CHUNK_NODES, node_body, 0)
            pltpu.async_copy(
                out_v.at[b],
                out_h.at[pl.ds(cg * CHUNK_NODES, CHUNK_NODES)],
                so[b])

        def wait_out(b, cg):
            pltpu.make_async_copy(
                out_v.at[b],
                out_h.at[pl.ds(cg * CHUNK_NODES, CHUNK_NODES)],
                so[b]).wait()

        def outer(it, carry):
            for b in range(2):
                c = it * 2 + b
                cg = chunk0 + c

                @pl.when(c < n)
                def _():
                    drain_gathers(b)

                    @pl.when(c + 2 < n)
                    def _():
                        pltpu.async_copy(idx_h.at[cg + 2], idx_v.at[b], si[b])

                    pltpu.make_async_copy(w_h.at[cg], w_v.at[b], sw[b]).wait()

                    @pl.when(c >= 2)
                    def _():
                        wait_out(b, cg - 2)

                    compute_chunk(b, cg)

                    @pl.when(c + 2 < n)
                    def _():
                        pltpu.make_async_copy(
                            idx_h.at[cg + 2], idx_v.at[b], si[b]).wait()
                        fire_gathers(b)
                        pltpu.async_copy(w_h.at[cg + 2], w_v.at[b], sw[b])

            return carry

        lax.fori_loop(0, MAXP, outer, 0)
        wait_out(0, chunk0)
        wait_out(1, chunk0)

    kern = pl.kernel(
        body,
        out_type=jax.ShapeDtypeStruct((NPAD, A32), jnp.int32),
        mesh=mesh,
        scratch_types=[
            pltpu.VMEM((2, IDX_ROWS, 128), jnp.int32),
            pltpu.VMEM((2, CHUNK_EDGES), jnp.float32),
            pltpu.VMEM((2, CHUNK_EDGES, row_w),
                       jnp.float32 if f32_in else jnp.int32),
            pltpu.VMEM((2, CHUNK_NODES, A32), jnp.int32),
        ] + [pltpu.SemaphoreType.DMA] * 8,
        compiler_params=pltpu.CompilerParams(use_tc_tiling_on_sc=False,
                                             needs_layout_passes=False),
    )
    return kern


def _make_pool_kernel(n_cry, ap):
    mesh = plsc.VectorSubcoreMesh(core_axis_name="c", subcore_axis_name="s",
                                  num_cores=2, num_subcores=16)
    cpw = n_cry // NW
    ipw = cpw * ap
    rows128 = ipw // 128

    def body(table_h, idx_h, out_h, idx_v, rows_v, out_v, sg):
        cid = lax.axis_index("c")
        sid = lax.axis_index("s")
        wid = sid * 2 + cid
        pltpu.sync_copy(idx_h.at[wid], idx_v)
        for r in range(rows128):
            pltpu.async_copy(
                table_h.at[idx_v.at[r]],
                rows_v.at[pl.ds(r * 128, 128)], sg)
        for r in range(rows128):
            pltpu.make_async_copy(
                table_h.at[idx_v.at[r]],
                rows_v.at[pl.ds(r * 128, 128)], sg).wait()
        inv = 1.0 / ap

        def cry_body(k, _):
            base = k * ap
            acc = [jnp.zeros((LANES,), jnp.float32) for _ in range(2 * (A32 // LANES))]
            for j in range(ap):
                for h in range(A32 // LANES):
                    lo, hi = _unpack_bf16(rows_v[base + j, pl.ds(h * LANES, LANES)])
                    acc[2 * h] = acc[2 * h] + lo
                    acc[2 * h + 1] = acc[2 * h + 1] + hi
            for i in range(A // LANES):
                out_v[k, pl.ds(i * LANES, LANES)] = acc[i] * inv
            return 0
        lax.fori_loop(0, cpw, cry_body, 0)
        pltpu.sync_copy(out_v, out_h.at[pl.ds(wid * cpw, cpw)])

    kern = pl.kernel(
        body,
        out_type=jax.ShapeDtypeStruct((n_cry, A), jnp.float32),
        mesh=mesh,
        scratch_types=[
            pltpu.VMEM((rows128, 128), jnp.int32),
            pltpu.VMEM((ipw, A32), jnp.int32),
            pltpu.VMEM((cpw, A), jnp.float32),
            pltpu.SemaphoreType.DMA,
        ],
        compiler_params=pltpu.CompilerParams(use_tc_tiling_on_sc=False,
                                             needs_layout_passes=False),
    )
    return kern


def _head_body(pool, wt0, wt1, wt2, wfc, bfc, wout, bout, out_o, h_o):
    wc = jnp.dot(jnp.dot(wt0[...], wt1[...], preferred_element_type=jnp.float32),
                 wt2[...], preferred_element_type=jnp.float32)
    crys = jnp.dot(pool[...].astype(jnp.float32), wc,
                   preferred_element_type=jnp.float32)
    pre = jnp.dot(crys, wfc[...], preferred_element_type=jnp.float32) + bfc[...]
    h = _softplus(pre)
    h_o[...] = h
    out_o[...] = jnp.dot(h, wout[...], preferred_element_type=jnp.float32) + bout[...]


def _head_call(pool, W_tp_0, W_tp_1, W_tp_2, W_fc, b_fc2, W_out, b_out2):
    n_cry = pool.shape[0]
    H = W_fc.shape[1]
    return pl.pallas_call(
        _head_body,
        out_shape=[
            jax.ShapeDtypeStruct((n_cry, 1), jnp.float32),
            jax.ShapeDtypeStruct((n_cry, H), jnp.float32),
        ],
    )(pool, W_tp_0, W_tp_1, W_tp_2, W_fc, b_fc2, W_out, b_out2)


def kernel(atom_fea, nbr_fea, nbr_idx, crystal_atom_idx, W_emb, b_emb,
           W_r1_0, b_r1_0, W_r2_0, b_r2_0, W_tp_0,
           W_r1_1, b_r1_1, W_r2_1, b_r2_1, W_tp_1,
           W_r1_2, b_r1_2, W_r2_2, b_r2_2, W_tp_2,
           W_fc, b_fc, W_out, b_out):
    n, m = nbr_idx.shape
    n_cry, ap = crystal_atom_idx.shape

    atom_t = atom_fea.T
    nbr_t3 = jnp.transpose(nbr_fea, (2, 1, 0))
    wr1cat = jnp.concatenate([W_r1_0, W_r1_1, W_r1_2], axis=1)
    br1cat = jnp.concatenate([b_r1_0, b_r1_1, b_r1_2]).reshape(1, 3 * NBR)
    w2sel = jnp.zeros((3 * NBR, 3), jnp.float32)
    w2sel = w2sel.at[0 * NBR:1 * NBR, 0].set(W_r2_0[:, 0])
    w2sel = w2sel.at[1 * NBR:2 * NBR, 1].set(W_r2_1[:, 0])
    w2sel = w2sel.at[2 * NBR:3 * NBR, 2].set(W_r2_2[:, 0])
    b2bc = jnp.broadcast_to(
        jnp.stack([b_r2_0[0], b_r2_1[0], b_r2_2[0]]).reshape(3, 1), (3, BE))
    b_emb2 = b_emb.reshape(1, A)

    x0, w3d = _prep_call(atom_t, nbr_t3, W_emb, b_emb2, wr1cat, br1cat, w2sel, b2bc)

    w_node = w3d.transpose(0, 2, 1).reshape(3, EPAD)
    idx_flat = nbr_idx.reshape(-1)
    idx_pad = jnp.pad(idx_flat, (0, EPAD - n * m)).reshape(NCHUNKS, IDX_ROWS, 128)
    w_chunks = [w_node[l].reshape(NCHUNKS, CHUNK_EDGES) for l in range(3)]

    agg_i = _make_agg_kernel(False)
    t = x0
    for l in range(3):
        t = agg_i(t, idx_pad, w_chunks[l])

    cry_idx = crystal_atom_idx.reshape(NW, (n_cry * ap) // (NW * 128), 128)
    pool = _make_pool_kernel(n_cry, ap)(t, cry_idx)

    out, h = _head_call(pool, W_tp_0, W_tp_1, W_tp_2, W_fc,
                        b_fc.reshape(1, -1), W_out, b_out.reshape(1, 1))
    return (out, h)
